# Initial kernel scaffold; baseline (speedup 1.0000x reference)
import jax
import jax.numpy as jnp
from jax.experimental import pallas as pl


def kernel(tokenized_prompts, token_embedding, ctx):
    raise NotImplementedError("write your pallas kernel here")


# speedup vs baseline: 1.0753x; 1.0753x over previous
import functools

import jax
import jax.numpy as jnp
from jax import lax
from jax.experimental import pallas as pl
from jax.experimental.pallas import tpu as pltpu
from jax.experimental.pallas import tpu_sc as plsc

N_CLS = 1000
SEQ = 77
D = 512
N_CTX = 16
N_SUF = SEQ - N_CTX - 1
NC, NS = 2, 16
NW = NC * NS
CPW = 32
G_ROWS = 64


def _make_sc_call():
    mesh = plsc.VectorSubcoreMesh(
        core_axis_name="c", subcore_axis_name="s", num_cores=NC, num_subcores=NS
    )

    @functools.partial(
        pl.kernel,
        mesh=mesh,
        out_type=jax.ShapeDtypeStruct((N_CLS, SEQ, D), jnp.float32),
        scratch_types=[
            pltpu.VMEM((CPW, G_ROWS), jnp.int32),
            pltpu.VMEM((N_CTX, D), jnp.float32),
            pltpu.VMEM((G_ROWS, D), jnp.float32),
            pltpu.SemaphoreType.DMA,
        ],
        compiler_params=pltpu.CompilerParams(use_tc_tiling_on_sc=False),
    )
    def sc_kernel(idx_hbm, table_hbm, ctx_hbm, out_hbm, idx_v, ctx_v, rows_v, sem):
        wid = lax.axis_index("s") * NC + lax.axis_index("c")
        pltpu.sync_copy(idx_hbm.at[wid], idx_v)
        pltpu.sync_copy(ctx_hbm, ctx_v)

        def body(c, carry):
            g = wid * CPW + c

            @pl.when(g < N_CLS)
            def _():
                pltpu.async_copy(table_hbm.at[idx_v.at[c]], rows_v, sem).wait()
                outg = out_hbm.at[g]
                pltpu.sync_copy(rows_v.at[pl.ds(0, 1)], outg.at[pl.ds(0, 1)])
                pltpu.sync_copy(ctx_v, outg.at[pl.ds(1, N_CTX)])
                pltpu.sync_copy(
                    rows_v.at[pl.ds(1, N_SUF)],
                    outg.at[pl.ds(1 + N_CTX, N_SUF)],
                )

            return carry

        lax.fori_loop(0, CPW, body, 0)

    return sc_kernel


_sc_call = _make_sc_call()


def kernel(tokenized_prompts, token_embedding, ctx):
    tok = tokenized_prompts.astype(jnp.int32)
    gidx = jnp.concatenate([tok[:, :1], tok[:, 1 + N_CTX:]], axis=1)
    gidx = jnp.pad(gidx, ((0, NW * CPW - N_CLS), (0, G_ROWS - gidx.shape[1])))
    gidx = gidx.reshape(NW, CPW, G_ROWS)
    return _sc_call(gidx, token_embedding, ctx)


# speedup vs baseline: 2.5391x; 2.3613x over previous
import functools

import jax
import jax.numpy as jnp
from jax import lax
from jax.experimental import pallas as pl
from jax.experimental.pallas import tpu as pltpu
from jax.experimental.pallas import tpu_sc as plsc

N_CLS = 1000
SEQ = 77
D = 512
N_CTX = 16
NA = 56
NB = 8
NTAIL = 5
NC, NS = 2, 16
NW = NC * NS
CPW = 32
LANES = 16


def _copy_row(src_ref, src_row, dst_ref, dst_row):
    for k in range(D // LANES):
        dst_ref[dst_row, pl.ds(k * LANES, LANES)] = (
            src_ref[src_row, pl.ds(k * LANES, LANES)]
        )


def _make_sc_call():
    mesh = plsc.VectorSubcoreMesh(
        core_axis_name="c", subcore_axis_name="s", num_cores=NC, num_subcores=NS
    )

    @functools.partial(
        pl.kernel,
        mesh=mesh,
        out_type=jax.ShapeDtypeStruct((N_CLS, SEQ, D), jnp.float32),
        scratch_types=[
            pltpu.VMEM((CPW, 1, NA), jnp.int32),
            pltpu.VMEM((CPW, 1, NB), jnp.int32),
            pltpu.VMEM((8, D), jnp.float32),
            pltpu.VMEM((SEQ, D), jnp.float32),
            pltpu.VMEM((NB, D), jnp.float32),
            pltpu.SemaphoreType.DMA,
        ],
    )
    def sc_kernel(idxa_hbm, idxb_hbm, table_hbm, cshift_hbm, out_hbm,
                  idxa_v, idxb_v, c15_v, blk_v, tl_v, sem):
        wid = lax.axis_index("s") * NC + lax.axis_index("c")
        pltpu.sync_copy(idxa_hbm.at[wid], idxa_v)
        pltpu.sync_copy(idxb_hbm.at[wid], idxb_v)
        pltpu.sync_copy(cshift_hbm, blk_v.at[pl.ds(0, 24)])
        pltpu.sync_copy(cshift_hbm.at[pl.ds(16, 8)], c15_v)

        def body(c, carry):
            g = wid * CPW + c

            @pl.when(g < N_CLS)
            def _():
                cpa = pltpu.async_copy(
                    table_hbm.at[idxa_v.at[c, 0]],
                    blk_v.at[pl.ds(N_CTX, NA)], sem,
                )
                cpb = pltpu.async_copy(table_hbm.at[idxb_v.at[c, 0]], tl_v, sem)
                cpa.wait()
                cpb.wait()
                _copy_row(blk_v, N_CTX, blk_v, 0)
                _copy_row(c15_v, 0, blk_v, N_CTX)
                for i in range(NTAIL):
                    _copy_row(tl_v, i, blk_v, N_CTX + NA + i)
                pltpu.sync_copy(blk_v, out_hbm.at[g])

            return carry

        lax.fori_loop(0, CPW, body, 0)

    return sc_kernel


_sc_call = _make_sc_call()


def kernel(tokenized_prompts, token_embedding, ctx):
    tok = tokenized_prompts.astype(jnp.int32)
    gidxa = jnp.concatenate([tok[:, :1], tok[:, 1 + N_CTX:1 + N_CTX + NA - 1]],
                            axis=1)
    gidxb = jnp.pad(tok[:, 1 + N_CTX + NA - 1:], ((0, 0), (0, NB - NTAIL)))
    gidxa = jnp.pad(gidxa, ((0, NW * CPW - N_CLS), (0, 0)))
    gidxb = jnp.pad(gidxb, ((0, NW * CPW - N_CLS), (0, 0)))
    gidxa = gidxa.reshape(NW, CPW, 1, NA)
    gidxb = gidxb.reshape(NW, CPW, 1, NB)
    cshift = jnp.pad(ctx, ((1, 7), (0, 0)))
    return _sc_call(gidxa, gidxb, token_embedding, cshift)
